# Initial kernel scaffold; baseline (speedup 1.0000x reference)
import jax
import jax.numpy as jnp
from jax.experimental import pallas as pl


def kernel(x, x_e, edge_index, in_norm_g, in_norm_b, in_proj_W, in_proj_b, e_norm_g, e_norm_b, e_proj_W, e_proj_b, gn_w, gn_b, gn_ms, hg_W, hg_b, skip_W, skip_b, gn_d_w, gn_d_b, gn_d_ms, hg_d_W, hg_d_b, skip_d_W, skip_d_b, fusion_W, fusion_b, lin_W, lin_b):
    raise NotImplementedError("write your pallas kernel here")


# speedup vs baseline: 1.2424x; 1.2424x over previous
import jax
import jax.numpy as jnp
from jax import lax
from jax.experimental import pallas as pl
from jax.experimental.pallas import tpu as pltpu
from jax.experimental.pallas import tpu_sc as plsc

N = 10000
NPAD = 10240
E = 320000
EPAD = 327680
H = 128
EPS = 1e-5

NC, NS = 2, 16
NW = NC * NS
EW = EPAD // NW
NB_E = EW // 128
RB = 1024
NRB = NPAD // RB
HR = NPAD // 128
CAP = 12288
TPW = NPAD // NW

_MESH = plsc.VectorSubcoreMesh(
    core_axis_name="c", subcore_axis_name="s", num_cores=NC, num_subcores=NS)


def _tc_a(xs_ref, g_ref, b_ref, W_ref, bias_ref, h0_ref, stat_ref, acc):
    br = pl.program_id(0)
    rb = pl.program_id(1)
    z = xs_ref[0]
    nrm = jnp.sqrt(jnp.sum(z * z, axis=1, keepdims=True))
    zl2 = z / jnp.maximum(nrm, 1e-12)
    z = jnp.where(br == 1, zl2, z)
    mu = jnp.mean(z, axis=1, keepdims=True)
    var = jnp.mean((z - mu) ** 2, axis=1, keepdims=True)
    zn = g_ref[0] * (z - mu) * lax.rsqrt(var + EPS) + b_ref[0]
    t = jnp.dot(zn, W_ref[0], preferred_element_type=jnp.float32) + bias_ref[0]
    h0 = jnp.where(t >= 0, t, 0.01 * t)
    h0_ref[0] = h0
    rows = rb * RB + lax.broadcasted_iota(jnp.int32, (RB, 1), 0)
    hm = jnp.where(rows < N, h0, 0.0)
    s1 = jnp.sum(hm, axis=0, keepdims=True)
    s2 = jnp.sum(hm * hm, axis=0, keepdims=True)

    @pl.when(rb == 0)
    def _():
        acc[...] = jnp.zeros_like(acc)

    acc[0:1] = acc[0:1] + s1
    acc[1:2] = acc[1:2] + s2

    @pl.when(rb == NRB - 1)
    def _():
        stat_ref[0] = acc[...]


def _tc_b(h0_ref, stat_ref, w_ref, b_ref, ms_ref, Whg_ref, Wsk_ref, bsk_ref,
          xw_ref, skip_ref):
    z = h0_ref[0]
    mu = stat_ref[0, 0:1] / N
    m2 = stat_ref[0, 1:2] / N
    ms = ms_ref[0]
    var = m2 - mu * mu * ms * (2.0 - ms)
    zg = w_ref[0] * (z - ms * mu) * lax.rsqrt(var + EPS) + b_ref[0]
    xw_ref[0] = jnp.dot(zg, Whg_ref[0], preferred_element_type=jnp.float32)
    skip_ref[0] = (jnp.dot(zg, Wsk_ref[0], preferred_element_type=jnp.float32)
                   + bsk_ref[0])


def _tc_b2(S1p_ref, S1dp_ref, Dvp_ref, Bep_ref, ef_ref, efd_ref,
           dinv_ref, binv_ref):
    Dv = jnp.sum(Dvp_ref[...], axis=0)
    Be = jnp.sum(Bep_ref[...], axis=0)
    dinv = jnp.where(Dv > 0, 1.0 / Dv, 0.0)
    binv = jnp.where(Be > 0, 1.0 / Be, 0.0)
    dinv_ref[...] = dinv
    binv_ref[...] = binv
    S1 = S1p_ref[0] + S1p_ref[1]
    S1d = S1dp_ref[0] + S1dp_ref[1]
    nr = RB // 128
    ef_ref[...] = (S1.reshape(nr, 128, 128) * binv[:, :, None]).reshape(RB, H)
    efd_ref[...] = (S1d.reshape(nr, 128, 128) * dinv[:, :, None]).reshape(RB, H)


def _tc_c(S2p_ref, S2dp_ref, dinv_ref, binv_ref, skip_ref, hgb_ref, hgdb_ref,
          h1_ref, he1_ref):
    dinv = dinv_ref[...]
    binv = binv_ref[...]
    S2 = S2p_ref[0] + S2p_ref[1]
    S2d = S2dp_ref[0] + S2dp_ref[1]
    nr = RB // 128
    co = (S2.reshape(nr, 128, 128) * dinv[:, :, None]).reshape(RB, H) + hgb_ref[...]
    cod = (S2d.reshape(nr, 128, 128) * binv[:, :, None]).reshape(RB, H) + hgdb_ref[...]
    h1_ref[...] = jnp.where(co >= 0, co, 0.01 * co) + skip_ref[0]
    he1_ref[...] = jnp.where(cod >= 0, cod, 0.01 * cod) + skip_ref[1]


def _tc_d(min_ref, he1_ref, Wt_ref, Wb_ref, fb_ref, Wl_ref, lb_ref, out_ref):
    m = min_ref[...]
    agg = jnp.where(jnp.isfinite(m), m, 0.0)
    fused = (jnp.dot(agg, Wt_ref[...], preferred_element_type=jnp.float32)
             + jnp.dot(he1_ref[...], Wb_ref[...], preferred_element_type=jnp.float32)
             + fb_ref[...])
    out_ref[...] = (jnp.dot(fused, Wl_ref[...], preferred_element_type=jnp.float32)
                    + lb_ref[...])


def _sc_segsum_body(gidx_hbm, sidx_hbm, table_hbm, zeros_hbm, out_hbm,
                    idxg_v, idxs_v, rows_v, acc_sh, sem):
    c = lax.axis_index("c")
    s = lax.axis_index("s")
    w = c * NS + s
    nz = NPAD // 128 // NS
    pltpu.sync_copy(zeros_hbm, rows_v)

    def zr(i, x):
        pltpu.sync_copy(rows_v, acc_sh.at[pl.ds((s * nz + i) * 128, 128)])
        return x

    lax.fori_loop(0, nz, zr, 0)
    plsc.subcore_barrier()

    def step(i, x):
        off = w * EW + i * 128
        pltpu.sync_copy(gidx_hbm.at[pl.ds(off, 128)], idxg_v)
        pltpu.sync_copy(sidx_hbm.at[pl.ds(off, 128)], idxs_v)
        pltpu.async_copy(table_hbm.at[idxg_v], rows_v, sem).wait()
        pltpu.sync_copy(rows_v, acc_sh.at[idxs_v], add=True)
        return x

    lax.fori_loop(0, NB_E, step, 0)
    plsc.subcore_barrier()

    def wr(i, x):
        r = (s * nz + i) * 128
        pltpu.sync_copy(acc_sh.at[pl.ds(r, 128)], rows_v)
        pltpu.sync_copy(rows_v, out_hbm.at[c].at[pl.ds(r, 128)])
        return x

    lax.fori_loop(0, nz, wr, 0)


def _sc_hist_body(srcp_hbm, dstp_hbm, zeros_hbm, outS_hbm, outD_hbm,
                  idx_v, ones_v, zb_v, histS_sh, histD_sh):
    c = lax.axis_index("c")
    s = lax.axis_index("s")
    w = c * NS + s
    nzs = NPAD // NS
    pltpu.sync_copy(zeros_hbm.at[pl.ds(0, nzs)], zb_v)
    pltpu.sync_copy(zb_v, histS_sh.at[pl.ds(s * nzs, nzs)])
    pltpu.sync_copy(zb_v, histD_sh.at[pl.ds(s * nzs, nzs)])
    for j in range(8):
        ones_v[pl.ds(j * 16, 16)] = jnp.full((16,), 1.0, jnp.float32)
    plsc.subcore_barrier()

    def batch(i, x):
        off = w * EW + i * 128
        pltpu.sync_copy(srcp_hbm.at[pl.ds(off, 128)], idx_v)
        pltpu.sync_copy(ones_v, histS_sh.at[idx_v], add=True)
        pltpu.sync_copy(dstp_hbm.at[pl.ds(off, 128)], idx_v)
        pltpu.sync_copy(ones_v, histD_sh.at[idx_v], add=True)
        return x

    lax.fori_loop(0, NB_E, batch, 0)
    plsc.subcore_barrier()
    pltpu.sync_copy(histS_sh.at[pl.ds(s * nzs, nzs)], zb_v)
    pltpu.sync_copy(zb_v, outS_hbm.at[c].at[pl.ds(s * nzs, nzs)])
    pltpu.sync_copy(histD_sh.at[pl.ds(s * nzs, nzs)], zb_v)
    pltpu.sync_copy(zb_v, outD_hbm.at[c].at[pl.ds(s * nzs, nzs)])


def _sc_min_body(srcp_hbm, dstp_hbm, h1_hbm, inf_hbm, safeS_hbm, safeD_hbm,
                 out_hbm, idxs_v, idxd_v, srcL_v, dstL_v, rows_v, table_v, sem):
    c = lax.axis_index("c")
    s = lax.axis_index("s")
    w = c * NS + s
    lo = w * TPW
    iota16 = lax.broadcasted_iota(jnp.int32, (16,), 0)

    pltpu.sync_copy(inf_hbm, table_v.at[pl.ds(0, 128)])
    pltpu.sync_copy(inf_hbm, table_v.at[pl.ds(128, 128)])
    pltpu.sync_copy(inf_hbm.at[pl.ds(0, TPW + 8 - 256)],
                    table_v.at[pl.ds(256, TPW + 8 - 256)])
    pltpu.sync_copy(safeS_hbm, srcL_v)
    pltpu.sync_copy(safeD_hbm, dstL_v)

    def fbatch(i, cnt):
        pltpu.sync_copy(dstp_hbm.at[pl.ds(i * 128, 128)], idxd_v)
        pltpu.sync_copy(srcp_hbm.at[pl.ds(i * 128, 128)], idxs_v)
        for j in range(8):
            dv = idxd_v[pl.ds(j * 16, 16)]
            sv = idxs_v[pl.ds(j * 16, 16)]
            dl = dv - lo
            m = (dl >= 0) & (dl < TPW)
            mi = jnp.where(m, 1, 0).astype(jnp.int32)
            csum = plsc.cumsum(mi)
            pos = jnp.minimum(cnt + csum - 1, CAP - 1)
            plsc.store_scatter(srcL_v, [pos], sv, mask=m)
            plsc.store_scatter(dstL_v, [pos], dl, mask=m)
            cnt = cnt + jnp.squeeze(lax.slice(csum, (15,), (16,)))
        return cnt

    cnt = lax.fori_loop(0, EPAD // 128, fbatch, jnp.int32(0))
    nb = (cnt + 127) // 128

    def gbatch(b, x):
        pltpu.async_copy(h1_hbm.at[srcL_v.at[pl.ds(b * 128, 128)]], rows_v,
                         sem).wait()

        def ed(e, y):
            dsp = plsc.load_gather(dstL_v, [iota16 * 0 + (b * 128 + e)])
            esp = iota16 * 0 + e
            for k in range(8):
                cix = k * 16 + iota16
                cur = plsc.load_gather(table_v, [dsp, cix])
                new = plsc.load_gather(rows_v, [esp, cix])
                plsc.store_scatter(table_v, [dsp, cix], jnp.minimum(cur, new))
            return y

        lax.fori_loop(0, 128, ed, 0)
        return x

    lax.fori_loop(0, nb, gbatch, 0)
    pltpu.sync_copy(table_v.at[pl.ds(0, TPW)], out_hbm.at[pl.ds(lo, TPW)])


_segsum = pl.kernel(
    _sc_segsum_body,
    out_type=jax.ShapeDtypeStruct((NC, NPAD, H), jnp.float32),
    mesh=_MESH,
    compiler_params=pltpu.CompilerParams(needs_layout_passes=False),
    scratch_types=[
        pltpu.VMEM((128,), jnp.int32),
        pltpu.VMEM((128,), jnp.int32),
        pltpu.VMEM((128, H), jnp.float32),
        pltpu.VMEM_SHARED((NPAD, H), jnp.float32),
        pltpu.SemaphoreType.DMA,
    ],
)

_hist = pl.kernel(
    _sc_hist_body,
    out_type=(jax.ShapeDtypeStruct((NC, NPAD), jnp.float32),
              jax.ShapeDtypeStruct((NC, NPAD), jnp.float32)),
    mesh=_MESH,
    compiler_params=pltpu.CompilerParams(needs_layout_passes=False),
    scratch_types=[
        pltpu.VMEM((128,), jnp.int32),
        pltpu.VMEM((128,), jnp.float32),
        pltpu.VMEM((NPAD // NS,), jnp.float32),
        pltpu.VMEM_SHARED((NPAD,), jnp.float32),
        pltpu.VMEM_SHARED((NPAD,), jnp.float32),
    ],
)

_segmin = pl.kernel(
    _sc_min_body,
    out_type=jax.ShapeDtypeStruct((NPAD, H), jnp.float32),
    mesh=_MESH,
    compiler_params=pltpu.CompilerParams(needs_layout_passes=False),
    scratch_types=[
        pltpu.VMEM((128,), jnp.int32),
        pltpu.VMEM((128,), jnp.int32),
        pltpu.VMEM((CAP,), jnp.int32),
        pltpu.VMEM((CAP,), jnp.int32),
        pltpu.VMEM((128, H), jnp.float32),
        pltpu.VMEM((TPW + 8, H), jnp.float32),
        pltpu.SemaphoreType.DMA,
    ],
)


def kernel(x, x_e, edge_index, in_norm_g, in_norm_b, in_proj_W, in_proj_b,
           e_norm_g, e_norm_b, e_proj_W, e_proj_b, gn_w, gn_b, gn_ms, hg_W,
           hg_b, skip_W, skip_b, gn_d_w, gn_d_b, gn_d_ms, hg_d_W, hg_d_b,
           skip_d_W, skip_d_b, fusion_W, fusion_b, lin_W, lin_b):
    f32 = jnp.float32
    src, dst = edge_index[0], edge_index[1]
    padi = (N + (jnp.arange(EPAD - E, dtype=jnp.int32) % 128)).astype(jnp.int32)
    srcp = jnp.concatenate([src, padi])
    dstp = jnp.concatenate([dst, padi])
    xs = jnp.stack([jnp.pad(x, ((0, NPAD - N), (0, 0))),
                    jnp.pad(x_e, ((0, NPAD - N), (0, 0)))])
    g2 = jnp.stack([in_norm_g, e_norm_g]).reshape(2, 1, H)
    b2 = jnp.stack([in_norm_b, e_norm_b]).reshape(2, 1, H)
    W2 = jnp.stack([in_proj_W, e_proj_W])
    bias2 = jnp.stack([in_proj_b, e_proj_b]).reshape(2, 1, H)

    h0, stat = pl.pallas_call(
        _tc_a,
        grid=(2, NRB),
        in_specs=[
            pl.BlockSpec((1, RB, H), lambda b, r: (b, r, 0)),
            pl.BlockSpec((1, 1, H), lambda b, r: (b, 0, 0)),
            pl.BlockSpec((1, 1, H), lambda b, r: (b, 0, 0)),
            pl.BlockSpec((1, H, H), lambda b, r: (b, 0, 0)),
            pl.BlockSpec((1, 1, H), lambda b, r: (b, 0, 0)),
        ],
        out_specs=[
            pl.BlockSpec((1, RB, H), lambda b, r: (b, r, 0)),
            pl.BlockSpec((1, 2, H), lambda b, r: (b, 0, 0)),
        ],
        out_shape=[
            jax.ShapeDtypeStruct((2, NPAD, H), f32),
            jax.ShapeDtypeStruct((2, 2, H), f32),
        ],
        scratch_shapes=[pltpu.VMEM((2, H), f32)],
    )(xs, g2, b2, W2, bias2)

    gw2 = jnp.stack([gn_w, gn_d_w]).reshape(2, 1, H)
    gb2 = jnp.stack([gn_b, gn_d_b]).reshape(2, 1, H)
    gms2 = jnp.stack([gn_ms, gn_d_ms]).reshape(2, 1, H)
    Whg2 = jnp.stack([hg_W, hg_d_W])
    Wsk2 = jnp.stack([skip_W, skip_d_W])
    bsk2 = jnp.stack([skip_b, skip_d_b]).reshape(2, 1, H)

    xw, skip = pl.pallas_call(
        _tc_b,
        grid=(2, NRB),
        in_specs=[
            pl.BlockSpec((1, RB, H), lambda b, r: (b, r, 0)),
            pl.BlockSpec((1, 2, H), lambda b, r: (b, 0, 0)),
            pl.BlockSpec((1, 1, H), lambda b, r: (b, 0, 0)),
            pl.BlockSpec((1, 1, H), lambda b, r: (b, 0, 0)),
            pl.BlockSpec((1, 1, H), lambda b, r: (b, 0, 0)),
            pl.BlockSpec((1, H, H), lambda b, r: (b, 0, 0)),
            pl.BlockSpec((1, H, H), lambda b, r: (b, 0, 0)),
            pl.BlockSpec((1, 1, H), lambda b, r: (b, 0, 0)),
        ],
        out_specs=[
            pl.BlockSpec((1, RB, H), lambda b, r: (b, r, 0)),
            pl.BlockSpec((1, RB, H), lambda b, r: (b, r, 0)),
        ],
        out_shape=[
            jax.ShapeDtypeStruct((2, NPAD, H), f32),
            jax.ShapeDtypeStruct((2, NPAD, H), f32),
        ],
    )(h0, stat, gw2, gb2, gms2, Whg2, Wsk2, bsk2)

    zeros128 = jnp.zeros((128, 128), f32)
    zerosN = jnp.zeros((NPAD,), f32)

    Dvp, Bep = _hist(srcp, dstp, zerosN)
    Dvp = Dvp.reshape(NC, HR, 128)
    Bep = Bep.reshape(NC, HR, 128)
    S1p = _segsum(srcp, dstp, xw[0], zeros128)
    S1dp = _segsum(dstp, srcp, xw[1], zeros128)

    ef, efd, dinv, binv = pl.pallas_call(
        _tc_b2,
        grid=(NRB,),
        in_specs=[
            pl.BlockSpec((2, RB, H), lambda r: (0, r, 0)),
            pl.BlockSpec((2, RB, H), lambda r: (0, r, 0)),
            pl.BlockSpec((NC, RB // 128, 128), lambda r: (0, r, 0)),
            pl.BlockSpec((NC, RB // 128, 128), lambda r: (0, r, 0)),
        ],
        out_specs=[
            pl.BlockSpec((RB, H), lambda r: (r, 0)),
            pl.BlockSpec((RB, H), lambda r: (r, 0)),
            pl.BlockSpec((RB // 128, 128), lambda r: (r, 0)),
            pl.BlockSpec((RB // 128, 128), lambda r: (r, 0)),
        ],
        out_shape=[
            jax.ShapeDtypeStruct((NPAD, H), f32),
            jax.ShapeDtypeStruct((NPAD, H), f32),
            jax.ShapeDtypeStruct((HR, 128), f32),
            jax.ShapeDtypeStruct((HR, 128), f32),
        ],
    )(S1p, S1dp, Dvp, Bep)

    S2p = _segsum(dstp, srcp, ef, zeros128)
    S2dp = _segsum(srcp, dstp, efd, zeros128)

    hgb = hg_b.reshape(1, H)
    hgdb = hg_d_b.reshape(1, H)
    h1, he1 = pl.pallas_call(
        _tc_c,
        grid=(NRB,),
        in_specs=[
            pl.BlockSpec((2, RB, H), lambda r: (0, r, 0)),
            pl.BlockSpec((2, RB, H), lambda r: (0, r, 0)),
            pl.BlockSpec((RB // 128, 128), lambda r: (r, 0)),
            pl.BlockSpec((RB // 128, 128), lambda r: (r, 0)),
            pl.BlockSpec((2, RB, H), lambda r: (0, r, 0)),
            pl.BlockSpec((1, H), lambda r: (0, 0)),
            pl.BlockSpec((1, H), lambda r: (0, 0)),
        ],
        out_specs=[
            pl.BlockSpec((RB, H), lambda r: (r, 0)),
            pl.BlockSpec((RB, H), lambda r: (r, 0)),
        ],
        out_shape=[
            jax.ShapeDtypeStruct((NPAD, H), f32),
            jax.ShapeDtypeStruct((NPAD, H), f32),
        ],
    )(S2p, S2dp, dinv, binv, skip, hgb, hgdb)

    inf128 = jnp.full((128, 128), jnp.inf, f32)
    safeS = jnp.full((CAP,), NPAD - 1, jnp.int32)
    safeD = jnp.full((CAP,), TPW, jnp.int32)
    minv = _segmin(srcp, dstp, h1, inf128, safeS, safeD)

    out = pl.pallas_call(
        _tc_d,
        grid=(NRB,),
        in_specs=[
            pl.BlockSpec((RB, H), lambda r: (r, 0)),
            pl.BlockSpec((RB, H), lambda r: (r, 0)),
            pl.BlockSpec((H, H), lambda r: (0, 0)),
            pl.BlockSpec((H, H), lambda r: (0, 0)),
            pl.BlockSpec((1, H), lambda r: (0, 0)),
            pl.BlockSpec((H, H), lambda r: (0, 0)),
            pl.BlockSpec((1, H), lambda r: (0, 0)),
        ],
        out_specs=pl.BlockSpec((RB, H), lambda r: (r, 0)),
        out_shape=jax.ShapeDtypeStruct((NPAD, H), f32),
    )(minv, he1, fusion_W[:H], fusion_W[H:], fusion_b.reshape(1, H),
      lin_W, lin_b.reshape(1, H))

    return out[:N]
